# Initial kernel scaffold; baseline (speedup 1.0000x reference)
import jax
import jax.numpy as jnp
from jax.experimental import pallas as pl


def kernel(x, edge_index, W1, b1, W2, b2):
    raise NotImplementedError("write your pallas kernel here")


# speedup vs baseline: 18.8835x; 18.8835x over previous
import functools

import jax
import jax.numpy as jnp
from jax import lax
from jax.experimental import pallas as pl
from jax.experimental.pallas import tpu as pltpu
from jax.experimental.pallas import tpu_sc as plsc

N = 10000
E = 160000
D = 256
H = 16
C = 10

NC = 2
NS = 16
L = 16
NW = NC * NS

CHUNK = 128
CPT = 40
EP = NW * CPT * CHUNK
ZR = 632
NP = NS * ZR
OR = ZR

_mesh = plsc.VectorSubcoreMesh(core_axis_name="c", subcore_axis_name="s")
_sc_params = pltpu.CompilerParams(use_tc_tiling_on_sc=False)


def _zero_acc(zbuf, acc, sid):
    def zrow(i, carry):
        zbuf[i, :] = jnp.zeros((L,), jnp.float32)
        return carry

    lax.fori_loop(0, ZR, zrow, 0)
    pltpu.sync_copy(zbuf, acc.at[pl.ds(sid * ZR, ZR)])


def _copy_out(acc, out_hbm, cid, sid):
    plsc.subcore_barrier()
    pltpu.sync_copy(
        acc.at[pl.ds(sid * OR, OR)],
        out_hbm.at[cid, pl.ds(sid * OR, OR)],
    )


@functools.partial(
    pl.kernel,
    out_type=jax.ShapeDtypeStruct((NC, NP, L), jnp.float32),
    mesh=_mesh,
    compiler_params=_sc_params,
    scratch_types=[
        pltpu.VMEM((CPT, CHUNK), jnp.int32),
        pltpu.VMEM((CHUNK, L), jnp.float32),
        pltpu.VMEM((ZR, L), jnp.float32),
        pltpu.VMEM_SHARED((NP, L), jnp.float32),
    ],
)
def _deg_kernel(dst_hbm, out_hbm, idx_d, ones, zbuf, acc):
    cid = lax.axis_index("c")
    sid = lax.axis_index("s")
    wid = cid * NS + sid
    _zero_acc(zbuf, acc, sid)

    def orow(i, carry):
        ones[i, :] = jnp.ones((L,), jnp.float32)
        return carry

    lax.fori_loop(0, CHUNK, orow, 0)
    pltpu.sync_copy(dst_hbm.at[pl.ds(wid * CPT, CPT)], idx_d)
    plsc.subcore_barrier()

    def chunk(j, carry):
        pltpu.sync_copy(ones, acc.at[idx_d.at[j]], add=True)
        return carry

    lax.fori_loop(0, CPT, chunk, 0)
    _copy_out(acc, out_hbm, cid, sid)


@functools.partial(
    pl.kernel,
    out_type=jax.ShapeDtypeStruct((NC, NP, L), jnp.float32),
    mesh=_mesh,
    compiler_params=_sc_params,
    scratch_types=[
        pltpu.VMEM((CPT, CHUNK), jnp.int32),
        pltpu.VMEM((CPT, CHUNK), jnp.int32),
        pltpu.VMEM((CHUNK, L), jnp.float32),
        pltpu.VMEM((ZR, L), jnp.float32),
        pltpu.VMEM_SHARED((NP, L), jnp.float32),
        pltpu.SemaphoreType.DMA,
    ],
)
def _agg_kernel(g_hbm, src_hbm, dst_hbm, out_hbm, idx_s, idx_d, rows, zbuf,
                acc, sem):
    cid = lax.axis_index("c")
    sid = lax.axis_index("s")
    wid = cid * NS + sid
    _zero_acc(zbuf, acc, sid)
    pltpu.sync_copy(src_hbm.at[pl.ds(wid * CPT, CPT)], idx_s)
    pltpu.sync_copy(dst_hbm.at[pl.ds(wid * CPT, CPT)], idx_d)
    plsc.subcore_barrier()

    def chunk(j, carry):
        pltpu.async_copy(g_hbm.at[idx_s.at[j]], rows, sem).wait()
        pltpu.sync_copy(rows, acc.at[idx_d.at[j]], add=True)
        return carry

    lax.fori_loop(0, CPT, chunk, 0)
    _copy_out(acc, out_hbm, cid, sid)


RB = 2000


def _tc1_body(x_ref, w1_ref, d0_ref, d1_ref, g1_ref, dinv_ref):
    deg = d0_ref[...] + d1_ref[...] + 1.0
    dinv = lax.rsqrt(deg)
    h = jnp.dot(x_ref[...], w1_ref[...], preferred_element_type=jnp.float32)
    g1_ref[...] = h * dinv
    dinv_ref[...] = dinv


def _tc2_body(a0_ref, a1_ref, g1_ref, dinv_ref, b1_ref, w2_ref, g2_ref):
    dinv = dinv_ref[...]
    out1 = dinv * (a0_ref[...] + a1_ref[...] + g1_ref[...]) + b1_ref[...]
    h2 = jnp.maximum(out1, 0.0)
    g2_ref[...] = jnp.dot(h2, w2_ref[...],
                          preferred_element_type=jnp.float32) * dinv


def _tc3_body(a0_ref, a1_ref, g2_ref, dinv_ref, b2_ref, out_ref):
    out_ref[...] = (dinv_ref[...] * (a0_ref[...] + a1_ref[...] + g2_ref[...])
                    + b2_ref[...])


def _row_spec(width):
    return pl.BlockSpec((RB, width), lambda i: (i, 0))


def _full_spec(shape):
    return pl.BlockSpec(shape, lambda i: tuple(0 for _ in shape))


_tc1 = pl.pallas_call(
    _tc1_body,
    grid=(N // RB,),
    in_specs=[_row_spec(D), _full_spec((D, H)), _row_spec(L), _row_spec(L)],
    out_specs=[_row_spec(L), _row_spec(L)],
    out_shape=[
        jax.ShapeDtypeStruct((N, L), jnp.float32),
        jax.ShapeDtypeStruct((N, L), jnp.float32),
    ],
)

_tc2 = pl.pallas_call(
    _tc2_body,
    grid=(N // RB,),
    in_specs=[_row_spec(L), _row_spec(L), _row_spec(L), _row_spec(L),
              _full_spec((1, L)), _full_spec((H, L))],
    out_specs=_row_spec(L),
    out_shape=jax.ShapeDtypeStruct((N, L), jnp.float32),
)

_tc3 = pl.pallas_call(
    _tc3_body,
    grid=(N // RB,),
    in_specs=[_row_spec(L), _row_spec(L), _row_spec(L), _row_spec(L),
              _full_spec((1, L))],
    out_specs=_row_spec(L),
    out_shape=jax.ShapeDtypeStruct((N, L), jnp.float32),
)


def kernel(x, edge_index, W1, b1, W2, b2):
    src = edge_index[0].astype(jnp.int32)
    dst = edge_index[1].astype(jnp.int32)
    pad = EP - E
    srcp = jnp.concatenate(
        [src, jnp.zeros((pad,), jnp.int32)]).reshape(NW * CPT, CHUNK)
    dstp = jnp.concatenate(
        [dst, jnp.full((pad,), N, jnp.int32)]).reshape(NW * CPT, CHUNK)

    degp = _deg_kernel(dstp)[:, :N]
    g1, dinv = _tc1(x, W1, degp[0], degp[1])
    acc1 = _agg_kernel(g1, srcp, dstp)[:, :N]

    w2p = jnp.pad(W2, ((0, 0), (0, L - C)))
    b1r = b1.reshape(1, H)
    b2p = jnp.pad(b2, (0, L - C)).reshape(1, L)
    g2 = _tc2(acc1[0], acc1[1], g1, dinv, b1r, w2p)
    acc2 = _agg_kernel(g2, srcp, dstp)[:, :N]
    out = _tc3(acc2[0], acc2[1], g2, dinv, b2p)
    return out[:, :C]


# speedup vs baseline: 26.0933x; 1.3818x over previous
import functools

import jax
import jax.numpy as jnp
from jax import lax
from jax.experimental import pallas as pl
from jax.experimental.pallas import tpu as pltpu
from jax.experimental.pallas import tpu_sc as plsc

N = 10000
E = 160000
D = 256
H = 16
C = 10

NC = 2
NS = 16
L = 16
NW = NC * NS

EPT = E // NW
ZR = 632
NP = NS * ZR
OR = ZR

_mesh = plsc.VectorSubcoreMesh(core_axis_name="c", subcore_axis_name="s")
_sc_params = pltpu.CompilerParams(use_tc_tiling_on_sc=False)


def _zero_acc(zbuf, acc, sid):
    def zrow(i, carry):
        zbuf[i, :] = jnp.zeros((L,), jnp.float32)
        return carry

    lax.fori_loop(0, ZR, zrow, 0)
    pltpu.sync_copy(zbuf, acc.at[pl.ds(sid * ZR, ZR)])


def _copy_out(acc, out_hbm, cid, sid):
    plsc.subcore_barrier()
    pltpu.sync_copy(
        acc.at[pl.ds(sid * OR, OR)],
        out_hbm.at[cid, pl.ds(sid * OR, OR)],
    )


@functools.partial(
    pl.kernel,
    out_type=jax.ShapeDtypeStruct((NC, NP, L), jnp.float32),
    mesh=_mesh,
    compiler_params=_sc_params,
    scratch_types=[
        pltpu.VMEM((EPT,), jnp.int32),
        pltpu.VMEM((EPT, L), jnp.float32),
        pltpu.VMEM((ZR, L), jnp.float32),
        pltpu.VMEM_SHARED((NP, L), jnp.float32),
    ],
)
def _deg_kernel(ei_hbm, out_hbm, idx_d, ones, zbuf, acc):
    cid = lax.axis_index("c")
    sid = lax.axis_index("s")
    wid = cid * NS + sid
    _zero_acc(zbuf, acc, sid)

    def orow(i, carry):
        ones[i, :] = jnp.ones((L,), jnp.float32)
        return carry

    lax.fori_loop(0, EPT, orow, 0)
    pltpu.sync_copy(ei_hbm.at[1, pl.ds(wid * EPT, EPT)], idx_d)
    plsc.subcore_barrier()
    pltpu.sync_copy(ones, acc.at[idx_d], add=True)
    _copy_out(acc, out_hbm, cid, sid)


@functools.partial(
    pl.kernel,
    out_type=jax.ShapeDtypeStruct((NC, NP, L), jnp.float32),
    mesh=_mesh,
    compiler_params=_sc_params,
    scratch_types=[
        pltpu.VMEM((EPT,), jnp.int32),
        pltpu.VMEM((EPT,), jnp.int32),
        pltpu.VMEM((EPT, L), jnp.float32),
        pltpu.VMEM((ZR, L), jnp.float32),
        pltpu.VMEM_SHARED((NP, L), jnp.float32),
        pltpu.SemaphoreType.DMA,
    ],
)
def _agg_kernel(g_hbm, ei_hbm, out_hbm, idx_s, idx_d, rows, zbuf, acc, sem):
    cid = lax.axis_index("c")
    sid = lax.axis_index("s")
    wid = cid * NS + sid
    _zero_acc(zbuf, acc, sid)
    pltpu.sync_copy(ei_hbm.at[0, pl.ds(wid * EPT, EPT)], idx_s)
    pltpu.sync_copy(ei_hbm.at[1, pl.ds(wid * EPT, EPT)], idx_d)
    plsc.subcore_barrier()
    pltpu.async_copy(g_hbm.at[idx_s], rows, sem).wait()
    pltpu.sync_copy(rows, acc.at[idx_d], add=True)
    _copy_out(acc, out_hbm, cid, sid)


RB = 2000


def _tc1_body(x_ref, w1_ref, d0_ref, d1_ref, g1_ref, dinv_ref):
    deg = d0_ref[...] + d1_ref[...] + 1.0
    dinv = lax.rsqrt(deg)
    h = jnp.dot(x_ref[...], w1_ref[...], preferred_element_type=jnp.float32)
    g1_ref[...] = h * dinv
    dinv_ref[...] = dinv


def _tc2_body(a0_ref, a1_ref, g1_ref, dinv_ref, b1_ref, w2_ref, g2_ref):
    dinv = dinv_ref[...]
    out1 = dinv * (a0_ref[...] + a1_ref[...] + g1_ref[...]) + b1_ref[...]
    h2 = jnp.maximum(out1, 0.0)
    g2_ref[...] = jnp.dot(h2, w2_ref[...],
                          preferred_element_type=jnp.float32) * dinv


def _tc3_body(a0_ref, a1_ref, g2_ref, dinv_ref, b2_ref, out_ref):
    out_ref[...] = (dinv_ref[...] * (a0_ref[...] + a1_ref[...] + g2_ref[...])
                    + b2_ref[...])


def _row_spec(width):
    return pl.BlockSpec((RB, width), lambda i: (i, 0))


def _full_spec(shape):
    return pl.BlockSpec(shape, lambda i: tuple(0 for _ in shape))


_tc1 = pl.pallas_call(
    _tc1_body,
    grid=(N // RB,),
    in_specs=[_row_spec(D), _full_spec((D, H)), _row_spec(L), _row_spec(L)],
    out_specs=[_row_spec(L), _row_spec(L)],
    out_shape=[
        jax.ShapeDtypeStruct((N, L), jnp.float32),
        jax.ShapeDtypeStruct((N, L), jnp.float32),
    ],
)

_tc2 = pl.pallas_call(
    _tc2_body,
    grid=(N // RB,),
    in_specs=[_row_spec(L), _row_spec(L), _row_spec(L), _row_spec(L),
              _full_spec((1, L)), _full_spec((H, L))],
    out_specs=_row_spec(L),
    out_shape=jax.ShapeDtypeStruct((N, L), jnp.float32),
)

_tc3 = pl.pallas_call(
    _tc3_body,
    grid=(N // RB,),
    in_specs=[_row_spec(L), _row_spec(L), _row_spec(L), _row_spec(L),
              _full_spec((1, L))],
    out_specs=_row_spec(L),
    out_shape=jax.ShapeDtypeStruct((N, L), jnp.float32),
)


def kernel(x, edge_index, W1, b1, W2, b2):
    ei = edge_index.astype(jnp.int32)

    degp = _deg_kernel(ei)[:, :N]
    g1, dinv = _tc1(x, W1, degp[0], degp[1])
    acc1 = _agg_kernel(g1, ei)[:, :N]

    w2p = jnp.pad(W2, ((0, 0), (0, L - C)))
    b1r = b1.reshape(1, H)
    b2p = jnp.pad(b2, (0, L - C)).reshape(1, L)
    g2 = _tc2(acc1[0], acc1[1], g1, dinv, b1r, w2p)
    acc2 = _agg_kernel(g2, ei)[:, :N]
    out = _tc3(acc2[0], acc2[1], g2, dinv, b2p)
    return out[:, :C]


# speedup vs baseline: 36.2563x; 1.3895x over previous
import functools

import jax
import jax.numpy as jnp
from jax import lax
from jax.experimental import pallas as pl
from jax.experimental.pallas import tpu as pltpu
from jax.experimental.pallas import tpu_sc as plsc

N = 10000
E = 160000
D = 256
H = 16
C = 10

NC = 2
NS = 16
L = 16
NW = NC * NS

EPT = E // NW
ZR = 632
NP = NS * ZR
OR = ZR

_mesh = plsc.VectorSubcoreMesh(core_axis_name="c", subcore_axis_name="s")
_sc_params = pltpu.CompilerParams(use_tc_tiling_on_sc=False)


def _fill_rows(buf, n_rows, value):
    def body(i, carry):
        for r in range(8):
            buf[i * 8 + r, :] = jnp.full((L,), value, jnp.float32)
        return carry

    lax.fori_loop(0, n_rows // 8, body, 0)


def _zero_acc(zbuf, acc, sid):
    _fill_rows(zbuf, ZR, 0.0)
    pltpu.sync_copy(zbuf, acc.at[pl.ds(sid * ZR, ZR)])


def _copy_out(acc, out_hbm, cid, sid):
    plsc.subcore_barrier()
    pltpu.sync_copy(
        acc.at[pl.ds(sid * OR, OR)],
        out_hbm.at[cid, pl.ds(sid * OR, OR)],
    )


@functools.partial(
    pl.kernel,
    out_type=jax.ShapeDtypeStruct((NC, NP, L), jnp.float32),
    mesh=_mesh,
    compiler_params=_sc_params,
    scratch_types=[
        pltpu.VMEM((EPT,), jnp.int32),
        pltpu.VMEM((EPT, L), jnp.float32),
        pltpu.VMEM((ZR, L), jnp.float32),
        pltpu.VMEM_SHARED((NP, L), jnp.float32),
    ],
)
def _deg_kernel(ei_hbm, out_hbm, idx_d, ones, zbuf, acc):
    cid = lax.axis_index("c")
    sid = lax.axis_index("s")
    wid = cid * NS + sid
    _zero_acc(zbuf, acc, sid)
    _fill_rows(ones, EPT, 1.0)
    pltpu.sync_copy(ei_hbm.at[1, pl.ds(wid * EPT, EPT)], idx_d)
    plsc.subcore_barrier()
    pltpu.sync_copy(ones, acc.at[idx_d], add=True)
    _copy_out(acc, out_hbm, cid, sid)


@functools.partial(
    pl.kernel,
    out_type=jax.ShapeDtypeStruct((NC, NP, L), jnp.float32),
    mesh=_mesh,
    compiler_params=_sc_params,
    scratch_types=[
        pltpu.VMEM((EPT,), jnp.int32),
        pltpu.VMEM((EPT,), jnp.int32),
        pltpu.VMEM((EPT, L), jnp.float32),
        pltpu.VMEM((ZR, L), jnp.float32),
        pltpu.VMEM_SHARED((NP, L), jnp.float32),
        pltpu.SemaphoreType.DMA,
    ],
)
def _agg_kernel(g_hbm, ei_hbm, out_hbm, idx_s, idx_d, rows, zbuf, acc, sem):
    cid = lax.axis_index("c")
    sid = lax.axis_index("s")
    wid = cid * NS + sid
    _zero_acc(zbuf, acc, sid)
    pltpu.sync_copy(ei_hbm.at[0, pl.ds(wid * EPT, EPT)], idx_s)
    pltpu.sync_copy(ei_hbm.at[1, pl.ds(wid * EPT, EPT)], idx_d)
    plsc.subcore_barrier()
    pltpu.async_copy(g_hbm.at[idx_s], rows, sem).wait()
    pltpu.sync_copy(rows, acc.at[idx_d], add=True)
    _copy_out(acc, out_hbm, cid, sid)


RB = 2000


def _tc1_body(x_ref, w1_ref, d0_ref, d1_ref, g1_ref, dinv_ref):
    deg = d0_ref[0] + d1_ref[0] + 1.0
    dinv = lax.rsqrt(deg)
    h = jnp.dot(x_ref[...], w1_ref[...], preferred_element_type=jnp.float32)
    g1_ref[...] = h * dinv
    dinv_ref[...] = dinv


def _tc2_body(a0_ref, a1_ref, g1_ref, dinv_ref, b1_ref, w2_ref, g2_ref):
    dinv = dinv_ref[...]
    out1 = dinv * (a0_ref[0] + a1_ref[0] + g1_ref[...]) + b1_ref[...]
    h2 = jnp.maximum(out1, 0.0)
    g2_ref[...] = jnp.dot(h2, w2_ref[...],
                          preferred_element_type=jnp.float32) * dinv


def _tc3_body(a0_ref, a1_ref, g2_ref, dinv_ref, b2_ref, out_ref):
    out_ref[...] = (dinv_ref[...] * (a0_ref[0] + a1_ref[0] + g2_ref[...])
                    + b2_ref[...])


def _row_spec(width):
    return pl.BlockSpec((RB, width), lambda i: (i, 0))


def _full_spec(shape):
    return pl.BlockSpec(shape, lambda i: tuple(0 for _ in shape))


def _part_spec(core):
    return pl.BlockSpec((1, RB, L), lambda i, c=core: (c, i, 0))


_tc1 = pl.pallas_call(
    _tc1_body,
    grid=(N // RB,),
    in_specs=[_row_spec(D), _full_spec((D, H)), _part_spec(0), _part_spec(1)],
    out_specs=[_row_spec(L), _row_spec(L)],
    out_shape=[
        jax.ShapeDtypeStruct((N, L), jnp.float32),
        jax.ShapeDtypeStruct((N, L), jnp.float32),
    ],
)

_tc2 = pl.pallas_call(
    _tc2_body,
    grid=(N // RB,),
    in_specs=[_part_spec(0), _part_spec(1), _row_spec(L), _row_spec(L),
              _full_spec((1, L)), _full_spec((H, L))],
    out_specs=_row_spec(L),
    out_shape=jax.ShapeDtypeStruct((N, L), jnp.float32),
)

_tc3 = pl.pallas_call(
    _tc3_body,
    grid=(N // RB,),
    in_specs=[_part_spec(0), _part_spec(1), _row_spec(L), _row_spec(L),
              _full_spec((1, L))],
    out_specs=_row_spec(L),
    out_shape=jax.ShapeDtypeStruct((N, L), jnp.float32),
)


def kernel(x, edge_index, W1, b1, W2, b2):
    ei = edge_index.astype(jnp.int32)

    degp = _deg_kernel(ei)
    g1, dinv = _tc1(x, W1, degp, degp)
    acc1 = _agg_kernel(g1, ei)

    w2p = jnp.pad(W2, ((0, 0), (0, L - C)))
    b1r = b1.reshape(1, H)
    b2p = jnp.pad(b2, (0, L - C)).reshape(1, L)
    g2 = _tc2(acc1, acc1, g1, dinv, b1r, w2p)
    acc2 = _agg_kernel(g2, ei)
    out = _tc3(acc2, acc2, g2, dinv, b2p)
    return out[:, :C]
